# baseline (speedup 1.0000x reference)
import functools

import jax
import jax.numpy as jnp
from jax import lax
from jax.experimental import pallas as pl
from jax.experimental.pallas import tpu as pltpu
from jax.experimental.pallas import tpu_sc as plsc

BATCH = 16384
EMB = 32
H0 = 128
H1 = 64

_NC = 2
_NS = 16
_NW = _NC * _NS
_BPW = BATCH // _NW


def _sc_gather_body(x0_hbm, x1_hbm, e0_hbm, e1_hbm, out0_hbm, out1_hbm,
                    idx0_v, idx1_v, rows0_v, rows1_v, sem0, sem1):
    wid = lax.axis_index("s") * _NC + lax.axis_index("c")
    base = wid * _BPW
    pltpu.sync_copy(x0_hbm.at[pl.ds(base, _BPW)], idx0_v)
    pltpu.sync_copy(x1_hbm.at[pl.ds(base, _BPW)], idx1_v)
    cp0 = pltpu.async_copy(e0_hbm.at[idx0_v], rows0_v, sem0)
    cp1 = pltpu.async_copy(e1_hbm.at[idx1_v], rows1_v, sem1)
    cp0.wait()
    cp1.wait()
    pltpu.sync_copy(rows0_v, out0_hbm.at[pl.ds(base, _BPW)])
    pltpu.sync_copy(rows1_v, out1_hbm.at[pl.ds(base, _BPW)])


_sc_gather = pl.kernel(
    _sc_gather_body,
    out_type=(
        jax.ShapeDtypeStruct((BATCH, EMB), jnp.float32),
        jax.ShapeDtypeStruct((BATCH, EMB), jnp.float32),
    ),
    mesh=plsc.VectorSubcoreMesh(core_axis_name="c", subcore_axis_name="s"),
    scratch_types=[
        pltpu.VMEM((_BPW,), jnp.int32),
        pltpu.VMEM((_BPW,), jnp.int32),
        pltpu.VMEM((_BPW, EMB), jnp.float32),
        pltpu.VMEM((_BPW, EMB), jnp.float32),
        pltpu.SemaphoreType.DMA,
        pltpu.SemaphoreType.DMA,
    ],
    compiler_params=pltpu.CompilerParams(use_tc_tiling_on_sc=False),
)


_BB = 2048


def _mlp_body(e0_ref, e1_ref, w0a_ref, w0b_ref, b0_ref, w1_ref, b1_ref,
              w2_ref, b2_ref, out_ref):
    h = jnp.dot(e0_ref[...], w0a_ref[...], preferred_element_type=jnp.float32)
    h += jnp.dot(e1_ref[...], w0b_ref[...], preferred_element_type=jnp.float32)
    h = jnp.maximum(h + b0_ref[...], 0.0)
    h = jnp.dot(h, w1_ref[...], preferred_element_type=jnp.float32)
    h = jnp.maximum(h + b1_ref[...], 0.0)
    out_ref[...] = jnp.dot(h, w2_ref[...],
                           preferred_element_type=jnp.float32) + b2_ref[...]


@jax.jit
def kernel(x, E0, E1, W0, b0, W1, b1, W2, b2):
    x0 = x[:, 0].astype(jnp.int32)
    x1 = x[:, 1].astype(jnp.int32)
    e0, e1 = _sc_gather(x0, x1, E0, E1)

    w0a = W0[:, :EMB].T
    w0b = W0[:, EMB:].T
    w1t = W1.T
    w2t = W2.T

    grid = BATCH // _BB
    out = pl.pallas_call(
        _mlp_body,
        grid=(grid,),
        in_specs=[
            pl.BlockSpec((_BB, EMB), lambda i: (i, 0)),
            pl.BlockSpec((_BB, EMB), lambda i: (i, 0)),
            pl.BlockSpec((EMB, H0), lambda i: (0, 0)),
            pl.BlockSpec((EMB, H0), lambda i: (0, 0)),
            pl.BlockSpec((1, H0), lambda i: (0, 0)),
            pl.BlockSpec((H0, H1), lambda i: (0, 0)),
            pl.BlockSpec((1, H1), lambda i: (0, 0)),
            pl.BlockSpec((H1, 1), lambda i: (0, 0)),
            pl.BlockSpec((1, 1), lambda i: (0, 0)),
        ],
        out_specs=pl.BlockSpec((_BB, 1), lambda i: (i, 0)),
        out_shape=jax.ShapeDtypeStruct((BATCH, 1), jnp.float32),
    )(e0, e1, w0a, w0b, b0.reshape(1, H0), w1t, b1.reshape(1, H1), w2t,
      b2.reshape(1, 1))
    return out
